# baseline (speedup 1.0000x reference)
import functools

import jax
import jax.numpy as jnp
from jax import lax
from jax.experimental import pallas as pl
from jax.experimental.pallas import tpu as pltpu
from jax.experimental.pallas import tpu_sc as plsc

N_NODES = 10000
D = 128
E_MSG = 256000
E_TGT = 64000

NC = 2
NS = 16
NW = NC * NS

CHUNK = 64
MSG_PER_W = 8192
TGT_PER_W = 2048
E_MSG_PAD = MSG_PER_W * NW
E_TGT_PAD = TGT_PER_W * NW
N_PAD = 10240
ROWS_PER_TILE = N_PAD // NS
W_W = 16
C_W = 16
DUMMY_DST = N_NODES
SHIFT = 14
MASK = (1 << SHIFT) - 1


def _sc_aggregate(x, msg_src, msg_dst, tgt_src, tgt_dst, tgt_w16):
    mesh = plsc.VectorSubcoreMesh(core_axis_name="c", subcore_axis_name="s")

    @functools.partial(
        pl.kernel,
        mesh=mesh,
        out_type=jax.ShapeDtypeStruct((NC, N_PAD, D), jnp.float32),
        scratch_types=[
            pltpu.VMEM((CHUNK,), jnp.int32),
            pltpu.VMEM((CHUNK,), jnp.int32),
            pltpu.VMEM((CHUNK,), jnp.int32),
            pltpu.VMEM((CHUNK,), jnp.int32),
            pltpu.VMEM((CHUNK, D), jnp.float32),
            pltpu.VMEM((CHUNK, D), jnp.float32),
            pltpu.VMEM((CHUNK, W_W), jnp.float32),
            pltpu.VMEM((CHUNK, W_W), jnp.float32),
            pltpu.VMEM_SHARED((N_PAD, D), jnp.float32),
            pltpu.SemaphoreType.DMA,
            pltpu.SemaphoreType.DMA,
            pltpu.SemaphoreType.DMA,
            pltpu.SemaphoreType.DMA,
        ],
    )
    def k(x_hbm, ms_hbm, md_hbm, ts_hbm, td_hbm, tw_hbm,
          sum_out,
          sidxA, didxA, sidxB, didxB, rowsA, rowsB,
          wbufA, wbufB, acc, sIA, sIB, sGA, sGB):
        cid = lax.axis_index("c")
        sid = lax.axis_index("s")
        wid = cid * NS + sid

        def fill_rows_zero(r, carry):
            for j in range(D // 16):
                rowsA[r, pl.ds(j * 16, 16)] = jnp.zeros((16,), jnp.float32)
            return carry
        lax.fori_loop(0, CHUNK, fill_rows_zero, 0)

        base_row = sid * ROWS_PER_TILE
        for t in range(ROWS_PER_TILE // CHUNK):
            r0 = base_row + t * CHUNK
            for g in range(CHUNK // 16):
                didxA[pl.ds(g * 16, 16)] = (
                    lax.iota(jnp.int32, 16) + (r0 + g * 16))
            pltpu.sync_copy(rowsA, acc.at[didxA])
        plsc.subcore_barrier()

        slots = {
            0: (sidxA, didxA, rowsA, wbufA, sIA, sGA),
            1: (sidxB, didxB, rowsB, wbufB, sIB, sGB),
        }

        def load(s, shbm, dhbm, whbm, b):
            si, di, _, wb, _, _ = slots[s]
            pltpu.sync_copy(shbm.at[pl.ds(b, CHUNK)], si)
            pltpu.sync_copy(dhbm.at[pl.ds(b, CHUNK)], di)
            if whbm is not None:
                pltpu.sync_copy(whbm.at[pl.ds(b, CHUNK)], wb)

        def gstart(s):
            si, _, rw, _, _, sG = slots[s]
            pltpu.async_copy(x_hbm.at[si], rw, sG)

        def gwait(s):
            si, _, rw, _, _, sG = slots[s]
            pltpu.make_async_copy(x_hbm.at[si], rw, sG).wait()

        def scale(s):
            _, _, rw, wb, _, _ = slots[s]

            def scale_row(r, c2):
                ws = wb[r, pl.ds(0, 16)]
                for j in range(D // 16):
                    rw[r, pl.ds(j * 16, 16)] = rw[r, pl.ds(j * 16, 16)] * ws
                return c2
            lax.fori_loop(0, CHUNK, scale_row, 0)

        def scat(s):
            _, di, rw, _, _, _ = slots[s]
            pltpu.sync_copy(rw, acc.at[di], add=True)

        def run_edges(shbm, dhbm, whbm, ebase, nchunks):
            npairs = nchunks // 2
            load(0, shbm, dhbm, whbm, ebase)
            gstart(0)

            def pair(j, carry):
                b1 = ebase + (2 * j + 1) * CHUNK
                b2 = ebase + jnp.minimum(2 * j + 2, nchunks - 1) * CHUNK
                load(1, shbm, dhbm, whbm, b1)
                gstart(1)
                gwait(0)
                if whbm is not None:
                    scale(0)
                scat(0)
                load(0, shbm, dhbm, whbm, b2)
                gstart(0)
                gwait(1)
                if whbm is not None:
                    scale(1)
                scat(1)
                return carry
            lax.fori_loop(0, npairs, pair, 0)
            gwait(0)

        run_edges(ms_hbm, md_hbm, None, wid * MSG_PER_W, MSG_PER_W // CHUNK)
        run_edges(ts_hbm, td_hbm, tw_hbm, wid * TGT_PER_W, TGT_PER_W // CHUNK)

        plsc.subcore_barrier()

        for t in range(ROWS_PER_TILE // CHUNK):
            r0 = base_row + t * CHUNK
            for g in range(CHUNK // 16):
                didxA[pl.ds(g * 16, 16)] = (
                    lax.iota(jnp.int32, 16) + (r0 + g * 16))
            pltpu.async_copy(acc.at[didxA], rowsA, sGA).wait()
            pltpu.sync_copy(rowsA, sum_out.at[cid, pl.ds(r0, CHUNK)])

    return k(x, msg_src, msg_dst, tgt_src, tgt_dst, tgt_w16)


def _sc_count128(ones8, msg_z, msg_dst, tgt_z, tgt_dst):
    mesh = plsc.VectorSubcoreMesh(core_axis_name="c", subcore_axis_name="s")

    @functools.partial(
        pl.kernel,
        mesh=mesh,
        out_type=jax.ShapeDtypeStruct((NC, N_PAD, D), jnp.float32),
        scratch_types=[
            pltpu.VMEM((CHUNK,), jnp.int32),
            pltpu.VMEM((CHUNK,), jnp.int32),
            pltpu.VMEM((CHUNK,), jnp.int32),
            pltpu.VMEM((CHUNK,), jnp.int32),
            pltpu.VMEM((CHUNK, D), jnp.float32),
            pltpu.VMEM((CHUNK, D), jnp.float32),
            pltpu.VMEM_SHARED((N_PAD, D), jnp.float32),
            pltpu.SemaphoreType.DMA,
            pltpu.SemaphoreType.DMA,
        ],
    )
    def k(x_hbm, ms_hbm, md_hbm, ts_hbm, td_hbm,
          cnt_out, sidxA, didxA, sidxB, didxB, rowsA, rowsB,
          acc, sGA, sGB):
        cid = lax.axis_index("c")
        sid = lax.axis_index("s")
        wid = cid * NS + sid

        def fill_rows_zero(r, carry):
            for j in range(D // 16):
                rowsA[r, pl.ds(j * 16, 16)] = jnp.zeros((16,), jnp.float32)
            return carry
        lax.fori_loop(0, CHUNK, fill_rows_zero, 0)

        base_row = sid * ROWS_PER_TILE
        for t in range(ROWS_PER_TILE // CHUNK):
            r0 = base_row + t * CHUNK
            for g in range(CHUNK // 16):
                didxA[pl.ds(g * 16, 16)] = (
                    lax.iota(jnp.int32, 16) + (r0 + g * 16))
            pltpu.sync_copy(rowsA, acc.at[didxA])
        plsc.subcore_barrier()

        slots = {
            0: (sidxA, didxA, rowsA, sGA),
            1: (sidxB, didxB, rowsB, sGB),
        }

        def load(s, shbm, dhbm, b):
            si, di, _, _ = slots[s]
            pltpu.sync_copy(shbm.at[pl.ds(b, CHUNK)], si)
            pltpu.sync_copy(dhbm.at[pl.ds(b, CHUNK)], di)

        def gstart(s):
            si, _, rw, sG = slots[s]
            pltpu.async_copy(x_hbm.at[si], rw, sG)

        def gwait(s):
            si, _, rw, sG = slots[s]
            pltpu.make_async_copy(x_hbm.at[si], rw, sG).wait()

        def scat(s):
            _, di, rw, _ = slots[s]
            pltpu.sync_copy(rw, acc.at[di], add=True)

        def run_edges(shbm, dhbm, ebase, nchunks):
            npairs = nchunks // 2
            load(0, shbm, dhbm, ebase)
            gstart(0)

            def pair(j, carry):
                b1 = ebase + (2 * j + 1) * CHUNK
                b2 = ebase + jnp.minimum(2 * j + 2, nchunks - 1) * CHUNK
                load(1, shbm, dhbm, b1)
                gstart(1)
                gwait(0)
                scat(0)
                load(0, shbm, dhbm, b2)
                gstart(0)
                gwait(1)
                scat(1)
                return carry
            lax.fori_loop(0, npairs, pair, 0)
            gwait(0)

        run_edges(ms_hbm, md_hbm, wid * MSG_PER_W, MSG_PER_W // CHUNK)
        run_edges(ts_hbm, td_hbm, wid * TGT_PER_W, TGT_PER_W // CHUNK)
        plsc.subcore_barrier()

        for t in range(ROWS_PER_TILE // CHUNK):
            r0 = base_row + t * CHUNK
            for g in range(CHUNK // 16):
                didxA[pl.ds(g * 16, 16)] = (
                    lax.iota(jnp.int32, 16) + (r0 + g * 16))
            pltpu.async_copy(acc.at[didxA], rowsA, sGA).wait()
            pltpu.sync_copy(rowsA, cnt_out.at[cid, pl.ds(r0, CHUNK)])

    return k(ones8, msg_z, msg_dst, tgt_z, tgt_dst)


BLK = 1000


def _tc_body(sum_ref, cnt_ref, x_ref, wrel_ref, brel_ref, wroot_ref, o_ref):
    s = sum_ref[0] + sum_ref[1]
    c = cnt_ref[0][:, 0:1] + cnt_ref[1][:, 0:1]
    mean = s / jnp.clip(c, 1.0, None)
    o_ref[...] = (
        jnp.dot(mean, wrel_ref[...], preferred_element_type=jnp.float32)
        + jnp.dot(x_ref[...], wroot_ref[...], preferred_element_type=jnp.float32)
        + brel_ref[...]
    )


def _tc_combine(sums, cnts, x, W_rel, b_rel, W_root):
    grid = (N_NODES // BLK,)
    return pl.pallas_call(
        _tc_body,
        grid=grid,
        in_specs=[
            pl.BlockSpec((NC, BLK, D), lambda i: (0, i, 0)),
            pl.BlockSpec((NC, BLK, D), lambda i: (0, i, 0)),
            pl.BlockSpec((BLK, D), lambda i: (i, 0)),
            pl.BlockSpec((D, D), lambda i: (0, 0)),
            pl.BlockSpec((1, D), lambda i: (0, 0)),
            pl.BlockSpec((D, D), lambda i: (0, 0)),
        ],
        out_specs=pl.BlockSpec((BLK, D), lambda i: (i, 0)),
        out_shape=jax.ShapeDtypeStruct((N_NODES, D), jnp.float32),
    )(sums, cnts, x, W_rel, b_rel, W_root)


def _pad(a, n_pad, fill):
    if n_pad:
        a = jnp.concatenate([a, jnp.full((n_pad,), fill, a.dtype)])
    return a


def kernel(x, message_edge_index, target_edge_index, target_edge_weights,
           W_rel, b_rel, W_root):
    msg_src = _pad(message_edge_index[0], E_MSG_PAD - E_MSG, 0)
    msg_dst = _pad(message_edge_index[1], E_MSG_PAD - E_MSG, DUMMY_DST)
    tgt_src = _pad(target_edge_index[0], E_TGT_PAD - E_TGT, 0)
    tgt_dst = _pad(target_edge_index[1], E_TGT_PAD - E_TGT, DUMMY_DST)
    tgt_w16 = jnp.broadcast_to(
        jnp.concatenate(
            [target_edge_weights, jnp.zeros((E_TGT_PAD - E_TGT,), jnp.float32)]
        )[:, None],
        (E_TGT_PAD, W_W),
    )

    sums = _sc_aggregate(x, msg_src, msg_dst, tgt_src, tgt_dst, tgt_w16)
    tok = (sums[0, 0, 0] * 0.0).astype(jnp.int32)
    ones_tbl = jnp.ones((N_NODES, D), jnp.float32)
    cnts = _sc_count128(ones_tbl, msg_src, msg_dst + tok, tgt_src,
                        tgt_dst + tok)
    out = _tc_combine(sums, cnts, x, W_rel, b_rel.reshape(1, D), W_root)
    return (out, target_edge_weights)


# speedup vs baseline: 1.2236x; 1.2236x over previous
import functools

import jax
import jax.numpy as jnp
from jax import lax
from jax.experimental import pallas as pl
from jax.experimental.pallas import tpu as pltpu
from jax.experimental.pallas import tpu_sc as plsc

N_NODES = 10000
D = 128
E_MSG = 256000
E_TGT = 64000

NC = 2
NS = 16
NW = NC * NS

CHUNK = 32
MSG_PER_W = E_MSG // NW
TGT_PER_W = 2048
E_MSG_PAD = E_MSG
E_TGT_PAD = TGT_PER_W * NW
N_PAD = 10240
ROWS_PER_TILE = N_PAD // NS
W_W = 16
C_W = 16
DUMMY_DST = N_NODES
SHIFT = 14
MASK = (1 << SHIFT) - 1


def _sc_aggregate(x, msg_src, msg_dst, tgt_src, tgt_dst, tgt_w16):
    mesh = plsc.VectorSubcoreMesh(core_axis_name="c", subcore_axis_name="s")

    @functools.partial(
        pl.kernel,
        mesh=mesh,
        out_type=jax.ShapeDtypeStruct((NC, N_PAD, D), jnp.float32),
        scratch_types=[
            pltpu.VMEM((CHUNK,), jnp.int32),
            pltpu.VMEM((CHUNK,), jnp.int32),
            pltpu.VMEM((CHUNK,), jnp.int32),
            pltpu.VMEM((CHUNK,), jnp.int32),
            pltpu.VMEM((CHUNK, D), jnp.float32),
            pltpu.VMEM((CHUNK, D), jnp.float32),
            pltpu.VMEM((CHUNK, W_W), jnp.float32),
            pltpu.VMEM((CHUNK, W_W), jnp.float32),
            pltpu.VMEM_SHARED((N_PAD, D), jnp.float32),
            pltpu.SemaphoreType.DMA,
            pltpu.SemaphoreType.DMA,
            pltpu.SemaphoreType.DMA,
            pltpu.SemaphoreType.DMA,
        ],
    )
    def k(x_hbm, ms_hbm, md_hbm, ts_hbm, td_hbm, tw_hbm,
          sum_out,
          sidxA, didxA, sidxB, didxB, rowsA, rowsB,
          wbufA, wbufB, acc, sIA, sIB, sGA, sGB):
        cid = lax.axis_index("c")
        sid = lax.axis_index("s")
        wid = cid * NS + sid

        def fill_rows_zero(r, carry):
            for j in range(D // 16):
                rowsA[r, pl.ds(j * 16, 16)] = jnp.zeros((16,), jnp.float32)
            return carry
        lax.fori_loop(0, CHUNK, fill_rows_zero, 0)

        base_row = sid * ROWS_PER_TILE
        for t in range(ROWS_PER_TILE // CHUNK):
            r0 = base_row + t * CHUNK
            for g in range(CHUNK // 16):
                didxA[pl.ds(g * 16, 16)] = (
                    lax.iota(jnp.int32, 16) + (r0 + g * 16))
            pltpu.sync_copy(rowsA, acc.at[didxA])
        plsc.subcore_barrier()

        slots = {
            0: (sidxA, didxA, rowsA, wbufA, sIA, sGA),
            1: (sidxB, didxB, rowsB, wbufB, sIB, sGB),
        }

        def load(s, shbm, dhbm, whbm, b):
            si, di, _, wb, _, _ = slots[s]
            pltpu.sync_copy(shbm.at[pl.ds(b, CHUNK)], si)
            pltpu.sync_copy(dhbm.at[pl.ds(b, CHUNK)], di)
            if whbm is not None:
                pltpu.sync_copy(whbm.at[pl.ds(b, CHUNK)], wb)

        def gstart(s):
            si, _, rw, _, _, sG = slots[s]
            pltpu.async_copy(x_hbm.at[si], rw, sG)

        def gwait(s):
            si, _, rw, _, _, sG = slots[s]
            pltpu.make_async_copy(x_hbm.at[si], rw, sG).wait()

        def scale(s):
            _, _, rw, wb, _, _ = slots[s]

            def scale_row(r, c2):
                ws = wb[r, pl.ds(0, 16)]
                for j in range(D // 16):
                    rw[r, pl.ds(j * 16, 16)] = rw[r, pl.ds(j * 16, 16)] * ws
                return c2
            lax.fori_loop(0, CHUNK, scale_row, 0)

        def scat(s):
            _, di, rw, _, _, _ = slots[s]
            pltpu.sync_copy(rw, acc.at[di], add=True)

        def run_edges(shbm, dhbm, whbm, ebase, nchunks):
            npairs = nchunks // 2
            load(0, shbm, dhbm, whbm, ebase)
            gstart(0)

            def pair(j, carry):
                b1 = ebase + (2 * j + 1) * CHUNK
                b2 = ebase + jnp.minimum(2 * j + 2, nchunks - 1) * CHUNK
                load(1, shbm, dhbm, whbm, b1)
                gstart(1)
                gwait(0)
                if whbm is not None:
                    scale(0)
                scat(0)
                load(0, shbm, dhbm, whbm, b2)
                gstart(0)
                gwait(1)
                if whbm is not None:
                    scale(1)
                scat(1)
                return carry
            lax.fori_loop(0, npairs, pair, 0)
            gwait(0)

        run_edges(ms_hbm, md_hbm, None, wid * MSG_PER_W, MSG_PER_W // CHUNK)
        run_edges(ts_hbm, td_hbm, tw_hbm, wid * TGT_PER_W, TGT_PER_W // CHUNK)

        plsc.subcore_barrier()

        for t in range(ROWS_PER_TILE // CHUNK):
            r0 = base_row + t * CHUNK
            for g in range(CHUNK // 16):
                didxA[pl.ds(g * 16, 16)] = (
                    lax.iota(jnp.int32, 16) + (r0 + g * 16))
            pltpu.async_copy(acc.at[didxA], rowsA, sGA).wait()
            pltpu.sync_copy(rowsA, sum_out.at[cid, pl.ds(r0, CHUNK)])

    return k(x, msg_src, msg_dst, tgt_src, tgt_dst, tgt_w16)


def _sc_count128(ones8, msg_z, msg_dst, tgt_z, tgt_dst):
    mesh = plsc.VectorSubcoreMesh(core_axis_name="c", subcore_axis_name="s")

    @functools.partial(
        pl.kernel,
        mesh=mesh,
        out_type=jax.ShapeDtypeStruct((NC, N_PAD, D), jnp.float32),
        scratch_types=[
            pltpu.VMEM((CHUNK,), jnp.int32),
            pltpu.VMEM((CHUNK,), jnp.int32),
            pltpu.VMEM((CHUNK,), jnp.int32),
            pltpu.VMEM((CHUNK,), jnp.int32),
            pltpu.VMEM((CHUNK, D), jnp.float32),
            pltpu.VMEM((CHUNK, D), jnp.float32),
            pltpu.VMEM_SHARED((N_PAD, D), jnp.float32),
            pltpu.SemaphoreType.DMA,
            pltpu.SemaphoreType.DMA,
        ],
    )
    def k(x_hbm, ms_hbm, md_hbm, ts_hbm, td_hbm,
          cnt_out, sidxA, didxA, sidxB, didxB, rowsA, rowsB,
          acc, sGA, sGB):
        cid = lax.axis_index("c")
        sid = lax.axis_index("s")
        wid = cid * NS + sid

        def fill_rows_zero(r, carry):
            for j in range(D // 16):
                rowsA[r, pl.ds(j * 16, 16)] = jnp.zeros((16,), jnp.float32)
            return carry
        lax.fori_loop(0, CHUNK, fill_rows_zero, 0)

        base_row = sid * ROWS_PER_TILE
        for t in range(ROWS_PER_TILE // CHUNK):
            r0 = base_row + t * CHUNK
            for g in range(CHUNK // 16):
                didxA[pl.ds(g * 16, 16)] = (
                    lax.iota(jnp.int32, 16) + (r0 + g * 16))
            pltpu.sync_copy(rowsA, acc.at[didxA])
        plsc.subcore_barrier()

        slots = {
            0: (sidxA, didxA, rowsA, sGA),
            1: (sidxB, didxB, rowsB, sGB),
        }

        def load(s, shbm, dhbm, b):
            si, di, _, _ = slots[s]
            pltpu.sync_copy(shbm.at[pl.ds(b, CHUNK)], si)
            pltpu.sync_copy(dhbm.at[pl.ds(b, CHUNK)], di)

        def gstart(s):
            si, _, rw, sG = slots[s]
            pltpu.async_copy(x_hbm.at[si], rw, sG)

        def gwait(s):
            si, _, rw, sG = slots[s]
            pltpu.make_async_copy(x_hbm.at[si], rw, sG).wait()

        def scat(s):
            _, di, rw, _ = slots[s]
            pltpu.sync_copy(rw, acc.at[di], add=True)

        def run_edges(shbm, dhbm, ebase, nchunks):
            npairs = nchunks // 2
            load(0, shbm, dhbm, ebase)
            gstart(0)

            def pair(j, carry):
                b1 = ebase + (2 * j + 1) * CHUNK
                b2 = ebase + jnp.minimum(2 * j + 2, nchunks - 1) * CHUNK
                load(1, shbm, dhbm, b1)
                gstart(1)
                gwait(0)
                scat(0)
                load(0, shbm, dhbm, b2)
                gstart(0)
                gwait(1)
                scat(1)
                return carry
            lax.fori_loop(0, npairs, pair, 0)
            gwait(0)

        run_edges(ms_hbm, md_hbm, wid * MSG_PER_W, MSG_PER_W // CHUNK)
        run_edges(ts_hbm, td_hbm, wid * TGT_PER_W, TGT_PER_W // CHUNK)
        plsc.subcore_barrier()

        for t in range(ROWS_PER_TILE // CHUNK):
            r0 = base_row + t * CHUNK
            for g in range(CHUNK // 16):
                didxA[pl.ds(g * 16, 16)] = (
                    lax.iota(jnp.int32, 16) + (r0 + g * 16))
            pltpu.async_copy(acc.at[didxA], rowsA, sGA).wait()
            pltpu.sync_copy(rowsA, cnt_out.at[cid, pl.ds(r0, CHUNK)])

    return k(ones8, msg_z, msg_dst, tgt_z, tgt_dst)


BLK = 1000


def _tc_body(sum_ref, cnt_ref, x_ref, wrel_ref, brel_ref, wroot_ref, o_ref):
    s = sum_ref[0] + sum_ref[1]
    c = cnt_ref[0][:, 0:1] + cnt_ref[1][:, 0:1]
    mean = s / jnp.clip(c, 1.0, None)
    o_ref[...] = (
        jnp.dot(mean, wrel_ref[...], preferred_element_type=jnp.float32)
        + jnp.dot(x_ref[...], wroot_ref[...], preferred_element_type=jnp.float32)
        + brel_ref[...]
    )


def _tc_combine(sums, cnts, x, W_rel, b_rel, W_root):
    grid = (N_NODES // BLK,)
    return pl.pallas_call(
        _tc_body,
        grid=grid,
        in_specs=[
            pl.BlockSpec((NC, BLK, D), lambda i: (0, i, 0)),
            pl.BlockSpec((NC, BLK, D), lambda i: (0, i, 0)),
            pl.BlockSpec((BLK, D), lambda i: (i, 0)),
            pl.BlockSpec((D, D), lambda i: (0, 0)),
            pl.BlockSpec((1, D), lambda i: (0, 0)),
            pl.BlockSpec((D, D), lambda i: (0, 0)),
        ],
        out_specs=pl.BlockSpec((BLK, D), lambda i: (i, 0)),
        out_shape=jax.ShapeDtypeStruct((N_NODES, D), jnp.float32),
    )(sums, cnts, x, W_rel, b_rel, W_root)


def _pad(a, n_pad, fill):
    if n_pad:
        a = jnp.concatenate([a, jnp.full((n_pad,), fill, a.dtype)])
    return a


def kernel(x, message_edge_index, target_edge_index, target_edge_weights,
           W_rel, b_rel, W_root):
    msg_src = message_edge_index[0]
    msg_dst = message_edge_index[1]
    tgt_src = _pad(target_edge_index[0], E_TGT_PAD - E_TGT, 0)
    tgt_dst = _pad(target_edge_index[1], E_TGT_PAD - E_TGT, DUMMY_DST)
    tgt_w16 = jnp.broadcast_to(
        jnp.concatenate(
            [target_edge_weights, jnp.zeros((E_TGT_PAD - E_TGT,), jnp.float32)]
        )[:, None],
        (E_TGT_PAD, W_W),
    )

    sums = _sc_aggregate(x, msg_src, msg_dst, tgt_src, tgt_dst, tgt_w16)
    tok = (sums[0, 0, 0] * 0.0).astype(jnp.int32)
    ones_tbl = jnp.ones((N_NODES, D), jnp.float32)
    cnts = _sc_count128(ones_tbl, msg_src, msg_dst + tok, tgt_src,
                        tgt_dst + tok)
    out = _tc_combine(sums, cnts, x, W_rel, b_rel.reshape(1, D), W_root)
    return (out, target_edge_weights)
